# Initial kernel scaffold; baseline (speedup 1.0000x reference)
import jax
import jax.numpy as jnp
from jax.experimental import pallas as pl


def kernel(x, W1, D1, g1, b1, W2, D2, g2, b2, W3, D3, g3, b3, W4, D4, g4, b4, W5, D5, g5, b5):
    raise NotImplementedError("write your pallas kernel here")


# speedup vs baseline: 1.5011x; 1.5011x over previous
import functools

import jax
import jax.numpy as jnp
from jax import lax
from jax.experimental import pallas as pl
from jax.experimental.pallas import tpu as pltpu

EPS = 1e-6
BN_EPS = 1e-5
NEG = 0.2
KNN = 20


def _bdiag(Wt, blk):
    r, O = Wt.shape
    out = jnp.zeros((3 * blk, 3 * O), jnp.float32)
    for d in range(3):
        out = out.at[d * blk:d * blk + r, d * O:(d + 1) * O].set(Wt)
    return out


def _p0_body(x_ref, xTc_ref, xr_ref, xrc_ref, xx_ref, xxr_ref,
             w2t_ref, d2t_ref, p_ref, d_ref, *, C, pad, K):
    xb = x_ref[0]
    xt = xTc_ref[0]
    xr = xr_ref[0]
    R = xr.shape[0]
    N = xb.shape[0]
    w2t = w2t_ref[...]
    d2t = d2t_ref[...]
    xx = xx_ref[0]
    xxr = xxr_ref[0]

    G = jnp.dot(xrc_ref[0], xt, preferred_element_type=jnp.float32)
    P0 = (2.0 * G - xx) - xxr
    iota = lax.broadcasted_iota(jnp.int32, (R, N), 1)
    neg_inf = jnp.float32(float("-inf"))

    def body(k, P):
        m = jnp.max(P, axis=1, keepdims=True)
        idx = jnp.min(jnp.where(P == m, iota, N), axis=1, keepdims=True)
        ohb = iota == idx
        Pn = jnp.where(ohb, neg_inf, P)
        e = jnp.dot(ohb.astype(jnp.float32), xb,
                    precision=lax.Precision.HIGHEST,
                    preferred_element_type=jnp.float32)
        em = e - xr
        if pad:
            zp = jnp.zeros((R, pad), jnp.float32)
            parts = [em[:, :C], xr[:, :C], zp,
                     em[:, C:2 * C], xr[:, C:2 * C], zp,
                     em[:, 2 * C:], xr[:, 2 * C:], zp]
        else:
            parts = [em[:, :C], xr[:, :C], em[:, C:2 * C], xr[:, C:2 * C],
                     em[:, 2 * C:], xr[:, 2 * C:]]
        feat = jnp.concatenate(parts, axis=1)
        p_ref[0, pl.ds(k * R, R), :] = jnp.dot(
            feat, w2t, preferred_element_type=jnp.float32)
        d_ref[0, pl.ds(k * R, R), :] = jnp.dot(
            feat, d2t, preferred_element_type=jnp.float32)
        return Pn

    lax.fori_loop(0, K, body, P0)


def _p1_body(p_ref, d_ref, mean_ref, var_ref, g_ref, b_ref, out_ref,
             *, O, K):
    meanv = mean_ref[...]
    denom = jnp.sqrt(var_ref[...] + BN_EPS)
    g = g_ref[...]
    b = b_ref[...]
    R = out_ref.shape[1]
    acc = None
    for k in range(K):
        pv = p_ref[0, k * R:(k + 1) * R, :]
        dv = d_ref[0, k * R:(k + 1) * R, :]
        p0 = pv[:, :O]
        p1 = pv[:, O:2 * O]
        p2 = pv[:, 2 * O:]
        n2 = p0 * p0 + p1 * p1 + p2 * p2
        norm = jnp.sqrt(n2) + EPS
        nbv = (norm - meanv) / denom * g + b
        q0 = (p0 / norm) * nbv
        q1 = (p1 / norm) * nbv
        q2 = (p2 / norm) * nbv
        d0 = dv[:, :O]
        d1 = dv[:, O:2 * O]
        d2 = dv[:, 2 * O:]
        dotv = q0 * d0 + q1 * d1 + q2 * d2
        den2 = d0 * d0 + d1 * d1 + d2 * d2
        pos = dotv >= 0.0
        cf = dotv / (den2 + EPS)
        o0 = NEG * q0 + (1.0 - NEG) * jnp.where(pos, q0, q0 - cf * d0)
        o1 = NEG * q1 + (1.0 - NEG) * jnp.where(pos, q1, q1 - cf * d1)
        o2 = NEG * q2 + (1.0 - NEG) * jnp.where(pos, q2, q2 - cf * d2)
        res = jnp.concatenate([o0, o1, o2], axis=1)
        acc = res if acc is None else acc + res
    out_ref[0] = acc * jnp.float32(1.0 / K)


def _edge_layer(xl, W, D, g, b):
    B, N, F = xl.shape
    C = W.shape[1] // 2
    O = W.shape[0]
    assert F == 3 * C
    R = min(128, N)
    NT = N // R
    blk = 2 * C if 6 * C <= 256 else 256
    pad = blk - 2 * C
    w2t = _bdiag(W.T, blk)
    d2t = _bdiag(D.T, blk)
    x_cm = xl.reshape(B, N, 3, C).transpose(0, 1, 3, 2).reshape(B, N, F)
    xTc = x_cm.transpose(0, 2, 1)
    xxc = jnp.sum(xTc ** 2, axis=1, keepdims=True)
    xxr_arr = xxc.transpose(0, 2, 1)

    p_out, d_out = pl.pallas_call(
        functools.partial(_p0_body, C=C, pad=pad, K=KNN),
        grid=(B, NT),
        in_specs=[
            pl.BlockSpec((1, N, F), lambda bb, t: (bb, 0, 0)),
            pl.BlockSpec((1, F, N), lambda bb, t: (bb, 0, 0)),
            pl.BlockSpec((1, R, F), lambda bb, t: (bb, t, 0)),
            pl.BlockSpec((1, R, F), lambda bb, t: (bb, t, 0)),
            pl.BlockSpec((1, 1, N), lambda bb, t: (bb, 0, 0)),
            pl.BlockSpec((1, R, 1), lambda bb, t: (bb, t, 0)),
            pl.BlockSpec((3 * blk, 3 * O), lambda bb, t: (0, 0)),
            pl.BlockSpec((3 * blk, 3 * O), lambda bb, t: (0, 0)),
        ],
        out_specs=[
            pl.BlockSpec((1, KNN * R, 3 * O), lambda bb, t: (bb * NT + t, 0, 0)),
            pl.BlockSpec((1, KNN * R, 3 * O), lambda bb, t: (bb * NT + t, 0, 0)),
        ],
        out_shape=[
            jax.ShapeDtypeStruct((B * NT, KNN * R, 3 * O), jnp.float32),
            jax.ShapeDtypeStruct((B * NT, KNN * R, 3 * O), jnp.float32),
        ],
    )(xl, xTc, xl, x_cm, xxc, xxr_arr, w2t, d2t)

    p5 = p_out.reshape(B, NT, KNN, R, 3, O)
    p5 = p5.transpose(0, 5, 4, 1, 3, 2).reshape(B, O, 3, N, KNN)
    p5 = jnp.einsum('oi,bi...->bo...', jnp.eye(O, dtype=jnp.float32), p5,
                    precision=lax.Precision.HIGHEST)
    normv = jnp.linalg.norm(p5, axis=2) + EPS
    meanv = jnp.mean(normv, axis=(0, 2, 3), keepdims=True)
    varv = jnp.var(normv, axis=(0, 2, 3), keepdims=True)

    out = pl.pallas_call(
        functools.partial(_p1_body, O=O, K=KNN),
        grid=(B, NT),
        in_specs=[
            pl.BlockSpec((1, KNN * R, 3 * O), lambda bb, t: (bb * NT + t, 0, 0)),
            pl.BlockSpec((1, KNN * R, 3 * O), lambda bb, t: (bb * NT + t, 0, 0)),
            pl.BlockSpec((1, O), lambda bb, t: (0, 0)),
            pl.BlockSpec((1, O), lambda bb, t: (0, 0)),
            pl.BlockSpec((1, O), lambda bb, t: (0, 0)),
            pl.BlockSpec((1, O), lambda bb, t: (0, 0)),
        ],
        out_specs=pl.BlockSpec((1, R, 3 * O), lambda bb, t: (bb, t, 0)),
        out_shape=jax.ShapeDtypeStruct((B, N, 3 * O), jnp.float32),
    )(p_out, d_out, meanv.reshape(1, O), varv.reshape(1, O),
      g.reshape(1, O), b.reshape(1, O))
    return out


def _f0_body(x_ref, wat_ref, dat_ref, p_ref, d_ref, *, C, pad):
    x_in = x_ref[0]
    if pad:
        zp = jnp.zeros((x_in.shape[0], pad), jnp.float32)
        xb = jnp.concatenate(
            [x_in[:, :C], zp, x_in[:, C:2 * C], zp, x_in[:, 2 * C:], zp],
            axis=1)
    else:
        xb = x_in
    p_ref[0] = jnp.dot(xb, wat_ref[...], preferred_element_type=jnp.float32)
    d_ref[0] = jnp.dot(xb, dat_ref[...], preferred_element_type=jnp.float32)


def _f1_body(p_ref, d_ref, mean_ref, var_ref, g_ref, b_ref, out_ref, *, O):
    meanv = mean_ref[...]
    denom = jnp.sqrt(var_ref[...] + BN_EPS)
    g = g_ref[...]
    b = b_ref[...]
    pv = p_ref[0]
    dv = d_ref[0]
    p0 = pv[:, :O]
    p1 = pv[:, O:2 * O]
    p2 = pv[:, 2 * O:]
    n2 = p0 * p0 + p1 * p1 + p2 * p2
    norm = jnp.sqrt(n2) + EPS
    nbv = (norm - meanv) / denom * g + b
    q0 = (p0 / norm) * nbv
    q1 = (p1 / norm) * nbv
    q2 = (p2 / norm) * nbv
    d0 = dv[:, 0:1]
    d1 = dv[:, 1:2]
    d2 = dv[:, 2:3]
    dotv = q0 * d0 + q1 * d1 + q2 * d2
    den2 = d0 * d0 + d1 * d1 + d2 * d2
    pos = dotv >= 0.0
    cf = dotv / (den2 + EPS)
    o0 = NEG * q0 + (1.0 - NEG) * jnp.where(pos, q0, q0 - cf * d0)
    o1 = NEG * q1 + (1.0 - NEG) * jnp.where(pos, q1, q1 - cf * d1)
    o2 = NEG * q2 + (1.0 - NEG) * jnp.where(pos, q2, q2 - cf * d2)
    out_ref[0] = jnp.concatenate([o0, o1, o2], axis=1)


def _final_layer(xl, W, D, g, b):
    B, N, F = xl.shape
    O = W.shape[0]
    C = W.shape[1]
    assert F == 3 * C
    blk = C if 3 * C <= 256 else 256
    pad = blk - C
    wat = _bdiag(W.T, blk)
    dat = _bdiag(D.T, blk)

    p_out, d_out = pl.pallas_call(
        functools.partial(_f0_body, C=C, pad=pad),
        grid=(B,),
        in_specs=[
            pl.BlockSpec((1, N, F), lambda bb: (bb, 0, 0)),
            pl.BlockSpec((3 * blk, 3 * O), lambda bb: (0, 0)),
            pl.BlockSpec((3 * blk, 3), lambda bb: (0, 0)),
        ],
        out_specs=[
            pl.BlockSpec((1, N, 3 * O), lambda bb: (bb, 0, 0)),
            pl.BlockSpec((1, N, 3), lambda bb: (bb, 0, 0)),
        ],
        out_shape=[
            jax.ShapeDtypeStruct((B, N, 3 * O), jnp.float32),
            jax.ShapeDtypeStruct((B, N, 3), jnp.float32),
        ],
    )(xl, wat, dat)

    p5 = p_out.reshape(B, N, 3, O).transpose(0, 3, 2, 1)
    p5 = jnp.einsum('oi,bi...->bo...', jnp.eye(O, dtype=jnp.float32), p5,
                    precision=lax.Precision.HIGHEST)
    normv = jnp.linalg.norm(p5, axis=2) + EPS
    meanv = jnp.mean(normv, axis=(0, 2), keepdims=True)
    varv = jnp.var(normv, axis=(0, 2), keepdims=True)

    out = pl.pallas_call(
        functools.partial(_f1_body, O=O),
        grid=(B,),
        in_specs=[
            pl.BlockSpec((1, N, 3 * O), lambda bb: (bb, 0, 0)),
            pl.BlockSpec((1, N, 3), lambda bb: (bb, 0, 0)),
            pl.BlockSpec((1, O), lambda bb: (0, 0)),
            pl.BlockSpec((1, O), lambda bb: (0, 0)),
            pl.BlockSpec((1, O), lambda bb: (0, 0)),
            pl.BlockSpec((1, O), lambda bb: (0, 0)),
        ],
        out_specs=pl.BlockSpec((1, N, 3 * O), lambda bb: (bb, 0, 0)),
        out_shape=jax.ShapeDtypeStruct((B, N, 3 * O), jnp.float32),
    )(p_out, d_out, meanv.reshape(1, O), varv.reshape(1, O),
      g.reshape(1, O), b.reshape(1, O))
    return out


def kernel(x, W1, D1, g1, b1, W2, D2, g2, b2, W3, D3, g3, b3,
           W4, D4, g4, b4, W5, D5, g5, b5):
    B, _, N = x.shape
    x0 = x.transpose(0, 2, 1)
    x1 = _edge_layer(x0, W1, D1, g1, b1)
    x2 = _edge_layer(x1, W2, D2, g2, b2)
    x3 = _edge_layer(x2, W3, D3, g3, b3)
    x4 = _edge_layer(x3, W4, D4, g4, b4)
    xc = jnp.concatenate([
        x1.reshape(B, N, 3, -1),
        x2.reshape(B, N, 3, -1),
        x3.reshape(B, N, 3, -1),
        x4.reshape(B, N, 3, -1),
    ], axis=-1).reshape(B, N, -1)
    x5 = _final_layer(xc, W5, D5, g5, b5)
    return x5.reshape(B, N, 3, -1).transpose(0, 3, 2, 1)
